# Initial kernel scaffold; baseline (speedup 1.0000x reference)
import jax
import jax.numpy as jnp
from jax.experimental import pallas as pl


def kernel(input_ids, table, W, b):
    raise NotImplementedError("write your pallas kernel here")


# speedup vs baseline: 7.6768x; 7.6768x over previous
import functools

import jax
import jax.numpy as jnp
from jax import lax
from jax.experimental import pallas as pl
from jax.experimental.pallas import tpu as pltpu
from jax.experimental.pallas import tpu_sc as plsc

_VOCAB = 100000
_HID = 128

_NC = 2
_NS = 16
_NW = _NC * _NS

_ROWS_BLK = 2000

_CHUNK = 128


def _linear_body(t_ref, w_ref, b_ref, o_ref):
    o_ref[...] = lax.dot_general(
        t_ref[...], w_ref[...],
        (((1,), (1,)), ((), ())),
        preferred_element_type=jnp.float32,
        precision=lax.Precision.HIGHEST,
    ) + b_ref[...]


def _transform_table(table, W, b):
    grid = _VOCAB // _ROWS_BLK
    return pl.pallas_call(
        _linear_body,
        grid=(grid,),
        in_specs=[
            pl.BlockSpec((_ROWS_BLK, _HID), lambda i: (i, 0)),
            pl.BlockSpec((_HID, _HID), lambda i: (0, 0)),
            pl.BlockSpec((1, _HID), lambda i: (0, 0)),
        ],
        out_specs=pl.BlockSpec((_ROWS_BLK, _HID), lambda i: (i, 0)),
        out_shape=jax.ShapeDtypeStruct((_VOCAB, _HID), jnp.float32),
    )(table, W, b.reshape(1, _HID))


def _make_gather(n_tokens):
    b_per_w = n_tokens // _NW
    n_chunks = b_per_w // _CHUNK
    mesh = plsc.VectorSubcoreMesh(core_axis_name="c", subcore_axis_name="s")

    @functools.partial(
        pl.kernel,
        mesh=mesh,
        out_type=jax.ShapeDtypeStruct((n_tokens, _HID), jnp.float32),
        scratch_types=[
            pltpu.VMEM((b_per_w,), jnp.int32),
            pltpu.VMEM((_CHUNK, _HID), jnp.float32),
            pltpu.VMEM((_CHUNK, _HID), jnp.float32),
            pltpu.SemaphoreType.DMA,
            pltpu.SemaphoreType.DMA,
        ],
    )
    def gather_k(table_hbm, idx_hbm, out_hbm, idx_v, rows0, rows1, sem0, sem1):
        wid = lax.axis_index("s") * _NC + lax.axis_index("c")
        base = wid * b_per_w
        pltpu.sync_copy(idx_hbm.at[pl.ds(base, b_per_w)], idx_v)

        def pair_body(p, _):
            i0 = p * 2
            i1 = i0 + 1
            g0 = pltpu.async_copy(
                table_hbm.at[idx_v.at[pl.ds(i0 * _CHUNK, _CHUNK)]], rows0, sem0)
            g1 = pltpu.async_copy(
                table_hbm.at[idx_v.at[pl.ds(i1 * _CHUNK, _CHUNK)]], rows1, sem1)
            g0.wait()
            pltpu.sync_copy(rows0, out_hbm.at[pl.ds(base + i0 * _CHUNK, _CHUNK)])
            g1.wait()
            pltpu.sync_copy(rows1, out_hbm.at[pl.ds(base + i1 * _CHUNK, _CHUNK)])
            return 0

        lax.fori_loop(0, n_chunks // 2, pair_body, 0)

    return gather_k


def kernel(input_ids, table, W, b):
    B, L = input_ids.shape
    n_tokens = B * L
    table2 = _transform_table(table, W, b)
    flat_idx = input_ids.reshape(n_tokens).astype(jnp.int32)
    out = _make_gather(n_tokens)(table2, flat_idx)
    return out.reshape(B, L, _HID)


# speedup vs baseline: 8.3415x; 1.0866x over previous
import functools

import jax
import jax.numpy as jnp
from jax import lax
from jax.experimental import pallas as pl
from jax.experimental.pallas import tpu as pltpu
from jax.experimental.pallas import tpu_sc as plsc

_VOCAB = 100000
_HID = 128

_NC = 2
_NS = 16
_NW = _NC * _NS

_ROWS_BLK = 2000

_CHUNK = 256


def _linear_body(t_ref, w_ref, b_ref, o_ref):
    o_ref[...] = lax.dot_general(
        t_ref[...], w_ref[...],
        (((1,), (1,)), ((), ())),
        preferred_element_type=jnp.float32,
        precision=lax.Precision.HIGHEST,
    ) + b_ref[...]


def _transform_table(table, W, b):
    grid = _VOCAB // _ROWS_BLK
    return pl.pallas_call(
        _linear_body,
        grid=(grid,),
        in_specs=[
            pl.BlockSpec((_ROWS_BLK, _HID), lambda i: (i, 0)),
            pl.BlockSpec((_HID, _HID), lambda i: (0, 0)),
            pl.BlockSpec((1, _HID), lambda i: (0, 0)),
        ],
        out_specs=pl.BlockSpec((_ROWS_BLK, _HID), lambda i: (i, 0)),
        out_shape=jax.ShapeDtypeStruct((_VOCAB, _HID), jnp.float32),
    )(table, W, b.reshape(1, _HID))


def _make_gather(n_tokens):
    b_per_w = n_tokens // _NW
    n_chunks = b_per_w // _CHUNK
    mesh = plsc.VectorSubcoreMesh(core_axis_name="c", subcore_axis_name="s")

    @functools.partial(
        pl.kernel,
        mesh=mesh,
        out_type=jax.ShapeDtypeStruct((n_tokens, _HID), jnp.float32),
        scratch_types=[
            pltpu.VMEM((b_per_w,), jnp.int32),
            pltpu.VMEM((_CHUNK, _HID), jnp.float32),
            pltpu.VMEM((_CHUNK, _HID), jnp.float32),
            pltpu.SemaphoreType.DMA,
            pltpu.SemaphoreType.DMA,
            pltpu.SemaphoreType.DMA,
            pltpu.SemaphoreType.DMA,
        ],
    )
    def gather_k(table_hbm, idx_hbm, out_hbm, idx_v, rows0, rows1,
                 sem0, sem1, osem0, osem1):
        wid = lax.axis_index("s") * _NC + lax.axis_index("c")
        base = wid * b_per_w
        pltpu.sync_copy(idx_hbm.at[pl.ds(base, b_per_w)], idx_v)

        def pair_body(p, _):
            i0 = p * 2
            i1 = i0 + 1
            g0 = pltpu.async_copy(
                table_hbm.at[idx_v.at[pl.ds(i0 * _CHUNK, _CHUNK)]], rows0, sem0)
            g1 = pltpu.async_copy(
                table_hbm.at[idx_v.at[pl.ds(i1 * _CHUNK, _CHUNK)]], rows1, sem1)
            g0.wait()
            w0 = pltpu.async_copy(
                rows0, out_hbm.at[pl.ds(base + i0 * _CHUNK, _CHUNK)], osem0)
            g1.wait()
            w1 = pltpu.async_copy(
                rows1, out_hbm.at[pl.ds(base + i1 * _CHUNK, _CHUNK)], osem1)
            w0.wait()
            w1.wait()
            return 0

        lax.fori_loop(0, n_chunks // 2, pair_body, 0)

    return gather_k


def kernel(input_ids, table, W, b):
    B, L = input_ids.shape
    n_tokens = B * L
    table2 = _transform_table(table, W, b)
    flat_idx = input_ids.reshape(n_tokens).astype(jnp.int32)
    out = _make_gather(n_tokens)(table2, flat_idx)
    return out.reshape(B, L, _HID)


# speedup vs baseline: 8.5711x; 1.0275x over previous
import functools

import jax
import jax.numpy as jnp
from jax import lax
from jax.experimental import pallas as pl
from jax.experimental.pallas import tpu as pltpu
from jax.experimental.pallas import tpu_sc as plsc

_VOCAB = 100000
_HID = 128

_NC = 2
_NS = 16
_NW = _NC * _NS

_ROWS_BLK = 2000

_CHUNK = 256


def _linear_body(t_ref, w_ref, b_ref, o_ref):
    o_ref[...] = lax.dot_general(
        t_ref[...], w_ref[...],
        (((1,), (1,)), ((), ())),
        preferred_element_type=jnp.float32,
        precision=lax.Precision.HIGHEST,
    ) + b_ref[...]


def _transform_table(table, W, b):
    grid = _VOCAB // _ROWS_BLK
    return pl.pallas_call(
        _linear_body,
        grid=(grid,),
        in_specs=[
            pl.BlockSpec((_ROWS_BLK, _HID), lambda i: (i, 0)),
            pl.BlockSpec((_HID, _HID), lambda i: (0, 0)),
            pl.BlockSpec((1, _HID), lambda i: (0, 0)),
        ],
        out_specs=pl.BlockSpec((_ROWS_BLK, _HID), lambda i: (i, 0)),
        out_shape=jax.ShapeDtypeStruct((_VOCAB, _HID), jnp.float32),
    )(table, W, b.reshape(1, _HID))


def _make_gather(n_tokens):
    b_per_w = n_tokens // _NW
    n_chunks = b_per_w // _CHUNK
    mesh = plsc.VectorSubcoreMesh(core_axis_name="c", subcore_axis_name="s")

    @functools.partial(
        pl.kernel,
        mesh=mesh,
        out_type=jax.ShapeDtypeStruct((n_tokens, _HID), jnp.float32),
        scratch_types=[
            pltpu.VMEM((b_per_w,), jnp.int32),
            pltpu.VMEM((_CHUNK, _HID), jnp.float32),
            pltpu.VMEM((_CHUNK, _HID), jnp.float32),
            pltpu.SemaphoreType.DMA,
            pltpu.SemaphoreType.DMA,
            pltpu.SemaphoreType.DMA,
            pltpu.SemaphoreType.DMA,
        ],
    )
    def gather_k(table_hbm, idx_hbm, out_hbm, idx_v, rows_a, rows_b,
                 gsem_a, gsem_b, osem_a, osem_b):
        wid = lax.axis_index("s") * _NC + lax.axis_index("c")
        base = wid * b_per_w
        pltpu.sync_copy(idx_hbm.at[pl.ds(base, b_per_w)], idx_v)

        def fire_gather(c, buf, sem):
            return pltpu.async_copy(
                table_hbm.at[idx_v.at[pl.ds(c * _CHUNK, _CHUNK)]], buf, sem)

        def fire_write(c, buf, sem):
            return pltpu.async_copy(
                buf, out_hbm.at[pl.ds(base + c * _CHUNK, _CHUNK)], sem)

        def body(p, _):
            c_a = 2 * p
            c_b = 2 * p + 1
            g_a = fire_gather(c_a, rows_a, gsem_a)

            @pl.when(p > 0)
            def _():
                pltpu.make_async_copy(
                    rows_b, out_hbm.at[pl.ds(base, _CHUNK)], osem_b).wait()

            g_a.wait()
            w_a = fire_write(c_a, rows_a, osem_a)
            g_b = fire_gather(c_b, rows_b, gsem_b)
            g_b.wait()
            fire_write(c_b, rows_b, osem_b)
            w_a.wait()
            return 0

        lax.fori_loop(0, n_chunks // 2, body, 0)
        pltpu.make_async_copy(
            rows_b, out_hbm.at[pl.ds(base, _CHUNK)], osem_b).wait()

    return gather_k


def kernel(input_ids, table, W, b):
    B, L = input_ids.shape
    n_tokens = B * L
    table2 = _transform_table(table, W, b)
    flat_idx = input_ids.reshape(n_tokens).astype(jnp.int32)
    out = _make_gather(n_tokens)(table2, flat_idx)
    return out.reshape(B, L, _HID)
